# baseline (speedup 1.0000x reference)
import functools

import jax
import jax.numpy as jnp
from jax import lax
from jax.experimental import pallas as pl
from jax.experimental.pallas import tpu as pltpu
from jax.experimental.pallas import tpu_sc as plsc

_N = 1000
_P = 1024
_L = 16
_NT = 16
_E = _P // _NT
_CPT = _E // _L
_SIGMA = 0.5
_THR = 0.05
_BIG_I32 = 2**31 - 1


def _snms_body(hx1, hy1, hx2, hy2, hm, out,
               vx1, vy1, vx2, vy2, var, vm, vfin, locb, gbuf, shared):
    @pl.when(lax.axis_index("c") == 0)
    def _():
        w = lax.axis_index("s")
        base = w * _E

        pltpu.sync_copy(hx1, vx1)
        pltpu.sync_copy(hy1, vy1)
        pltpu.sync_copy(hx2, vx2)
        pltpu.sync_copy(hy2, vy2)
        pltpu.sync_copy(hm, vm)

        lanes = lax.iota(jnp.int32, _L)
        lane0 = lanes == 0
        stride8 = lanes * 8
        dnums = lax.GatherDimensionNumbers(
            offset_dims=(), collapsed_slice_dims=(0,), start_index_map=(0,))

        def perm(x, idx):
            return lax.gather(x, idx[:, None], dnums, (1,),
                              mode=lax.GatherScatterMode.PROMISE_IN_BOUNDS)

        def bcast_max(x):
            for sh in (8, 4, 2, 1):
                x = jnp.maximum(x, perm(x, lanes ^ sh))
            return x

        def bcast_min_i32(x):
            for sh in (8, 4, 2, 1):
                x = jnp.minimum(x, perm(x, lanes ^ sh))
            return x

        zeros = jnp.zeros((_L,), jnp.float32)
        for k in range(_CPT):
            sl = pl.ds(base + k * _L, _L)
            var[sl] = (vx2[sl] - vx1[sl]) * (vy2[sl] - vy1[sl])
            vfin[sl] = zeros

        def local_scan(decayed):
            b1 = jnp.full((_L,), -2.0, jnp.float32)
            i1 = jnp.zeros((_L,), jnp.int32)
            b2 = jnp.full((_L,), -3.0, jnp.float32)
            i2 = jnp.zeros((_L,), jnp.int32)
            for k in range(_CPT):
                off = base + k * _L
                sl = pl.ds(off, _L)
                mc = vm[sl]
                if decayed is None:
                    mn = mc
                else:
                    bx1, by1, bx2, by2, a_i = decayed
                    cx1 = vx1[sl]
                    cy1 = vy1[sl]
                    cx2 = vx2[sl]
                    cy2 = vy2[sl]
                    xx1 = jnp.where(bx1 > cx1, bx1, cx1)
                    yy1 = jnp.where(by1 > cy1, by1, cy1)
                    xx2 = jnp.where(bx2 < cx2, bx2, cx2)
                    yy2 = jnp.where(by2 < cy2, by2, cy2)
                    dx = xx2 - xx1
                    dy = yy2 - yy1
                    inter = (jnp.where(dx > 0.0, dx, 0.0)
                             * jnp.where(dy > 0.0, dy, 0.0))
                    iou = inter / (a_i + var[sl] - inter + 1e-7)
                    dec = jnp.exp(iou * iou * (-1.0 / _SIGMA))
                    mn = jnp.where(mc >= 0.0, mc * dec, mc)
                    vm[sl] = mn
                ix = lanes + off
                gt1 = mn > b1
                gt2 = mn > b2
                b2 = jnp.where(gt1, b1, jnp.where(gt2, mn, b2))
                i2 = jnp.where(gt1, i1, jnp.where(gt2, ix, i2))
                b1 = jnp.where(gt1, mn, b1)
                i1 = jnp.where(gt1, ix, i1)
            lv1 = bcast_max(b1)
            li1 = bcast_min_i32(jnp.where(b1 == lv1, i1, _BIG_I32))
            isw = i1 == li1
            ev = jnp.where(isw, b2, b1)
            ei = jnp.where(isw, i2, i1)
            lv2 = bcast_max(ev)
            li2 = bcast_min_i32(jnp.where(ev == lv2, ei, _BIG_I32))
            return lv1, li1, lv2, li2

        def exchange(par, lv1, li1, lv2, li2):
            locb[:] = jnp.where(
                lane0, lv1,
                jnp.where(lanes == 1, plsc.bitcast(li1, jnp.float32),
                          jnp.where(lanes == 2, lv2,
                                    plsc.bitcast(li2, jnp.float32))))
            pltpu.sync_copy(locb.at[pl.ds(0, 8)],
                            shared.at[par & 1, pl.ds(w * 8, 8)])
            plsc.subcore_barrier()
            pltpu.sync_copy(shared.at[par & 1], gbuf)
            v1 = plsc.load_gather(gbuf, [stride8])
            x1 = plsc.bitcast(plsc.load_gather(gbuf, [stride8 + 1]), jnp.int32)
            v2 = plsc.load_gather(gbuf, [stride8 + 2])
            x2 = plsc.bitcast(plsc.load_gather(gbuf, [stride8 + 3]), jnp.int32)
            g1v = bcast_max(v1)
            g1i = bcast_min_i32(jnp.where(v1 == g1v, x1, _BIG_I32))
            isw = x1 == g1i
            ev = jnp.where(isw, v2, v1)
            ei = jnp.where(isw, x2, x1)
            g2v = bcast_max(ev)
            g2i = bcast_min_i32(jnp.where(ev == g2v, ei, _BIG_I32))
            ax1 = plsc.load_gather(vx1, [g1i])
            ay1 = plsc.load_gather(vy1, [g1i])
            ax2 = plsc.load_gather(vx2, [g1i])
            ay2 = plsc.load_gather(vy2, [g1i])
            qx1 = plsc.load_gather(vx1, [g2i])
            qy1 = plsc.load_gather(vy1, [g2i])
            qx2 = plsc.load_gather(vx2, [g2i])
            qy2 = plsc.load_gather(vy2, [g2i])
            ox = jnp.where(ax2 < qx2, ax2, qx2) - jnp.where(ax1 > qx1, ax1, qx1)
            oy = jnp.where(ay2 < qy2, ay2, qy2) - jnp.where(ay1 > qy1, ay1, qy1)
            disjoint = jnp.any((ox <= 0.0) | (oy <= 0.0))
            return g1i, g1v, g2i, g2v, disjoint, par + 1

        t2 = local_scan(None)
        bo, v, pbo, pv, pvalid, par = exchange(0, *t2)

        def body(t, carry):
            bo, v, pbo, pv, pvalid, par = carry
            plsc.store_scatter(vfin, [bo], v, mask=lane0)
            plsc.store_scatter(vm, [bo], jnp.full((_L,), -1.0, jnp.float32),
                               mask=lane0)
            bx1 = plsc.load_gather(vx1, [bo])
            by1 = plsc.load_gather(vy1, [bo])
            bx2 = plsc.load_gather(vx2, [bo])
            by2 = plsc.load_gather(vy2, [bo])
            a_i = (bx2 - bx1) * (by2 - by1)
            t2 = local_scan((bx1, by1, bx2, by2, a_i))

            def take_pending():
                return pbo, pv, pbo, pv, jnp.zeros((), jnp.bool_), par

            def do_exchange():
                return exchange(par, *t2)

            return lax.cond(pvalid, take_pending, do_exchange)

        lax.fori_loop(0, _N, body, (bo, v, pbo, pv, pvalid, par))

        for k in range(_CPT):
            sl = pl.ds(base + k * _L, _L)
            f = vfin[sl]
            vfin[sl] = jnp.where(f >= _THR, f, 0.0)
        pltpu.sync_copy(vfin.at[pl.ds(base, _E)], out.at[pl.ds(base, _E)])


_snms = functools.partial(
    pl.kernel,
    out_type=jax.ShapeDtypeStruct((_P,), jnp.float32),
    mesh=plsc.VectorSubcoreMesh(core_axis_name="c", subcore_axis_name="s",
                                num_cores=2, num_subcores=16),
    scratch_types=(
        [pltpu.VMEM((_P,), jnp.float32) for _ in range(7)]
        + [pltpu.VMEM((_L,), jnp.float32),
           pltpu.VMEM((_NT * 8,), jnp.float32),
           pltpu.VMEM_SHARED((2, _NT * 8), jnp.float32)]
    ),
    compiler_params=pltpu.CompilerParams(needs_layout_passes=False),
)(_snms_body)


@jax.jit
def kernel(boxes, scores):
    pad = _P - _N
    return _snms(
        jnp.pad(boxes[:, 0], (0, pad)),
        jnp.pad(boxes[:, 1], (0, pad)),
        jnp.pad(boxes[:, 2], (0, pad)),
        jnp.pad(boxes[:, 3], (0, pad)),
        jnp.pad(scores, (0, pad), constant_values=-1.0),
    )[:_N]


# speedup vs baseline: 1.0484x; 1.0484x over previous
import functools

import jax
import jax.numpy as jnp
from jax import lax
from jax.experimental import pallas as pl
from jax.experimental.pallas import tpu as pltpu
from jax.experimental.pallas import tpu_sc as plsc

_N = 1000
_P = 1024
_L = 16
_NT = 16
_E = _P // _NT
_CPT = _E // _L
_SIGMA = 0.5
_THR = 0.05
_BIG_I32 = 2**31 - 1


def _snms_body(hx1, hy1, hx2, hy2, hm, out,
               vx1, vy1, vx2, vy2, var, vm, vfin, locb, gbuf, shared):
    @pl.when(lax.axis_index("c") == 0)
    def _():
        w = lax.axis_index("s")
        base = w * _E

        pltpu.sync_copy(hx1, vx1)
        pltpu.sync_copy(hy1, vy1)
        pltpu.sync_copy(hx2, vx2)
        pltpu.sync_copy(hy2, vy2)
        pltpu.sync_copy(hm, vm)

        lanes = lax.iota(jnp.int32, _L)
        lane0 = lanes == 0
        stride8 = lanes * 8
        dnums = lax.GatherDimensionNumbers(
            offset_dims=(), collapsed_slice_dims=(0,), start_index_map=(0,))

        def perm(x, idx):
            return lax.gather(x, idx[:, None], dnums, (1,),
                              mode=lax.GatherScatterMode.PROMISE_IN_BOUNDS)

        lane15 = jnp.full((_L,), 15, jnp.int32)

        def bcast_max(x):
            return perm(plsc.cummax(x), lane15)

        def bcast_min_i32(x):
            return -perm(plsc.cummax(-x), lane15)

        zeros = jnp.zeros((_L,), jnp.float32)
        for k in range(_CPT):
            sl = pl.ds(base + k * _L, _L)
            var[sl] = (vx2[sl] - vx1[sl]) * (vy2[sl] - vy1[sl])
            vfin[sl] = zeros

        def decay_chunk(sl, decayed):
            bx1, by1, bx2, by2, a_i = decayed
            mc = vm[sl]
            cx1 = vx1[sl]
            cy1 = vy1[sl]
            cx2 = vx2[sl]
            cy2 = vy2[sl]
            xx1 = jnp.where(bx1 > cx1, bx1, cx1)
            yy1 = jnp.where(by1 > cy1, by1, cy1)
            xx2 = jnp.where(bx2 < cx2, bx2, cx2)
            yy2 = jnp.where(by2 < cy2, by2, cy2)
            dx = xx2 - xx1
            dy = yy2 - yy1
            inter = (jnp.where(dx > 0.0, dx, 0.0)
                     * jnp.where(dy > 0.0, dy, 0.0))
            iou = inter / (a_i + var[sl] - inter + 1e-7)
            dec = jnp.exp(iou * iou * (-1.0 / _SIGMA))
            mn = jnp.where(mc >= 0.0, mc * dec, mc)
            vm[sl] = mn
            return mn

        def decay_only(decayed):
            for k in range(_CPT):
                decay_chunk(pl.ds(base + k * _L, _L), decayed)

        def local_scan(decayed):
            b1 = jnp.full((_L,), -2.0, jnp.float32)
            i1 = jnp.zeros((_L,), jnp.int32)
            b2 = jnp.full((_L,), -3.0, jnp.float32)
            i2 = jnp.zeros((_L,), jnp.int32)
            for k in range(_CPT):
                off = base + k * _L
                sl = pl.ds(off, _L)
                if decayed is None:
                    mn = vm[sl]
                else:
                    mn = decay_chunk(sl, decayed)
                ix = lanes + off
                gt1 = mn > b1
                gt2 = mn > b2
                b2 = jnp.where(gt1, b1, jnp.where(gt2, mn, b2))
                i2 = jnp.where(gt1, i1, jnp.where(gt2, ix, i2))
                b1 = jnp.where(gt1, mn, b1)
                i1 = jnp.where(gt1, ix, i1)
            lv1 = bcast_max(b1)
            li1 = bcast_min_i32(jnp.where(b1 == lv1, i1, _BIG_I32))
            isw = i1 == li1
            ev = jnp.where(isw, b2, b1)
            ei = jnp.where(isw, i2, i1)
            lv2 = bcast_max(ev)
            li2 = bcast_min_i32(jnp.where(ev == lv2, ei, _BIG_I32))
            return lv1, li1, lv2, li2

        def exchange(par, lv1, li1, lv2, li2):
            locb[:] = jnp.where(
                lane0, lv1,
                jnp.where(lanes == 1, plsc.bitcast(li1, jnp.float32),
                          jnp.where(lanes == 2, lv2,
                                    plsc.bitcast(li2, jnp.float32))))
            pltpu.sync_copy(locb.at[pl.ds(0, 8)],
                            shared.at[par & 1, pl.ds(w * 8, 8)])
            plsc.subcore_barrier()
            pltpu.sync_copy(shared.at[par & 1], gbuf)
            v1 = plsc.load_gather(gbuf, [stride8])
            x1 = plsc.bitcast(plsc.load_gather(gbuf, [stride8 + 1]), jnp.int32)
            v2 = plsc.load_gather(gbuf, [stride8 + 2])
            x2 = plsc.bitcast(plsc.load_gather(gbuf, [stride8 + 3]), jnp.int32)
            g1v = bcast_max(v1)
            g1i = bcast_min_i32(jnp.where(v1 == g1v, x1, _BIG_I32))
            isw = x1 == g1i
            ev = jnp.where(isw, v2, v1)
            ei = jnp.where(isw, x2, x1)
            g2v = bcast_max(ev)
            g2i = bcast_min_i32(jnp.where(ev == g2v, ei, _BIG_I32))
            ax1 = plsc.load_gather(vx1, [g1i])
            ay1 = plsc.load_gather(vy1, [g1i])
            ax2 = plsc.load_gather(vx2, [g1i])
            ay2 = plsc.load_gather(vy2, [g1i])
            qx1 = plsc.load_gather(vx1, [g2i])
            qy1 = plsc.load_gather(vy1, [g2i])
            qx2 = plsc.load_gather(vx2, [g2i])
            qy2 = plsc.load_gather(vy2, [g2i])
            ox = jnp.where(ax2 < qx2, ax2, qx2) - jnp.where(ax1 > qx1, ax1, qx1)
            oy = jnp.where(ay2 < qy2, ay2, qy2) - jnp.where(ay1 > qy1, ay1, qy1)
            disjoint = jnp.any((ox <= 0.0) | (oy <= 0.0))
            return g1i, g1v, g2i, g2v, disjoint, par + 1

        t2 = local_scan(None)
        bo, v, pbo, pv, pvalid, par = exchange(0, *t2)

        def body(t, carry):
            bo, v, pbo, pv, pvalid, par = carry
            plsc.store_scatter(vfin, [bo], v, mask=lane0)
            plsc.store_scatter(vm, [bo], jnp.full((_L,), -1.0, jnp.float32),
                               mask=lane0)
            bx1 = plsc.load_gather(vx1, [bo])
            by1 = plsc.load_gather(vy1, [bo])
            bx2 = plsc.load_gather(vx2, [bo])
            by2 = plsc.load_gather(vy2, [bo])
            a_i = (bx2 - bx1) * (by2 - by1)
            dargs = (bx1, by1, bx2, by2, a_i)

            def take_pending():
                decay_only(dargs)
                return pbo, pv, pbo, pv, jnp.zeros((), jnp.bool_), par

            def do_exchange():
                return exchange(par, *local_scan(dargs))

            return lax.cond(pvalid, take_pending, do_exchange)

        lax.fori_loop(0, _N, body, (bo, v, pbo, pv, pvalid, par))

        for k in range(_CPT):
            sl = pl.ds(base + k * _L, _L)
            f = vfin[sl]
            vfin[sl] = jnp.where(f >= _THR, f, 0.0)
        pltpu.sync_copy(vfin.at[pl.ds(base, _E)], out.at[pl.ds(base, _E)])


_snms = functools.partial(
    pl.kernel,
    out_type=jax.ShapeDtypeStruct((_P,), jnp.float32),
    mesh=plsc.VectorSubcoreMesh(core_axis_name="c", subcore_axis_name="s",
                                num_cores=2, num_subcores=16),
    scratch_types=(
        [pltpu.VMEM((_P,), jnp.float32) for _ in range(7)]
        + [pltpu.VMEM((_L,), jnp.float32),
           pltpu.VMEM((_NT * 8,), jnp.float32),
           pltpu.VMEM_SHARED((2, _NT * 8), jnp.float32)]
    ),
    compiler_params=pltpu.CompilerParams(needs_layout_passes=False),
)(_snms_body)


@jax.jit
def kernel(boxes, scores):
    pad = _P - _N
    return _snms(
        jnp.pad(boxes[:, 0], (0, pad)),
        jnp.pad(boxes[:, 1], (0, pad)),
        jnp.pad(boxes[:, 2], (0, pad)),
        jnp.pad(boxes[:, 3], (0, pad)),
        jnp.pad(scores, (0, pad), constant_values=-1.0),
    )[:_N]


# speedup vs baseline: 1.1397x; 1.0871x over previous
import functools

import jax
import jax.numpy as jnp
from jax import lax
from jax.experimental import pallas as pl
from jax.experimental.pallas import tpu as pltpu
from jax.experimental.pallas import tpu_sc as plsc

_N = 1000
_P = 1024
_L = 16
_NT = 16
_E = _P // _NT
_CPT = _E // _L
_SIGMA = 0.5
_THR = 0.05
_BIG_I32 = 2**31 - 1


def _snms_body(hx1, hy1, hx2, hy2, hm, out,
               vx1, vy1, vx2, vy2, var, vm, vfin, locb, gbuf, shared):
    @pl.when(lax.axis_index("c") == 0)
    def _():
        w = lax.axis_index("s")
        base = w * _E

        pltpu.sync_copy(hx1, vx1)
        pltpu.sync_copy(hy1, vy1)
        pltpu.sync_copy(hx2, vx2)
        pltpu.sync_copy(hy2, vy2)
        pltpu.sync_copy(hm, vm)

        lanes = lax.iota(jnp.int32, _L)
        lane0 = lanes == 0
        stride8 = lanes * 8
        dnums = lax.GatherDimensionNumbers(
            offset_dims=(), collapsed_slice_dims=(0,), start_index_map=(0,))

        def perm(x, idx):
            return lax.gather(x, idx[:, None], dnums, (1,),
                              mode=lax.GatherScatterMode.PROMISE_IN_BOUNDS)

        lane15 = jnp.full((_L,), 15, jnp.int32)

        def bcast_max(x):
            return perm(plsc.cummax(x), lane15)

        def bcast_min_i32(x):
            return -perm(plsc.cummax(-x), lane15)

        zeros = jnp.zeros((_L,), jnp.float32)
        for k in range(_CPT):
            sl = pl.ds(base + k * _L, _L)
            var[sl] = (vx2[sl] - vx1[sl]) * (vy2[sl] - vy1[sl])
            vfin[sl] = zeros

        def decay_chunk(sl, decayed):
            bx1, by1, bx2, by2, a_i = decayed
            mc = vm[sl]
            cx1 = vx1[sl]
            cy1 = vy1[sl]
            cx2 = vx2[sl]
            cy2 = vy2[sl]
            xx1 = jnp.where(bx1 > cx1, bx1, cx1)
            yy1 = jnp.where(by1 > cy1, by1, cy1)
            xx2 = jnp.where(bx2 < cx2, bx2, cx2)
            yy2 = jnp.where(by2 < cy2, by2, cy2)
            dx = xx2 - xx1
            dy = yy2 - yy1
            inter = (jnp.where(dx > 0.0, dx, 0.0)
                     * jnp.where(dy > 0.0, dy, 0.0))
            iou = inter / (a_i + var[sl] - inter + 1e-7)
            dec = jnp.exp(iou * iou * (-1.0 / _SIGMA))
            mn = jnp.where(mc >= 0.0, mc * dec, mc)
            vm[sl] = mn
            return mn

        def decay_only(decayed):
            for k in range(_CPT):
                decay_chunk(pl.ds(base + k * _L, _L), decayed)

        def local_scan(decayed):
            b1 = jnp.full((_L,), -2.0, jnp.float32)
            i1 = jnp.zeros((_L,), jnp.int32)
            b2 = jnp.full((_L,), -3.0, jnp.float32)
            i2 = jnp.zeros((_L,), jnp.int32)
            for k in range(_CPT):
                off = base + k * _L
                sl = pl.ds(off, _L)
                if decayed is None:
                    mn = vm[sl]
                else:
                    mn = decay_chunk(sl, decayed)
                ix = lanes + off
                gt1 = mn > b1
                gt2 = mn > b2
                b2 = jnp.where(gt1, b1, jnp.where(gt2, mn, b2))
                i2 = jnp.where(gt1, i1, jnp.where(gt2, ix, i2))
                b1 = jnp.where(gt1, mn, b1)
                i1 = jnp.where(gt1, ix, i1)
            lv1 = bcast_max(b1)
            li1 = bcast_min_i32(jnp.where(b1 == lv1, i1, _BIG_I32))
            isw = i1 == li1
            ev = jnp.where(isw, b2, b1)
            ei = jnp.where(isw, i2, i1)
            lv2 = bcast_max(ev)
            li2 = bcast_min_i32(jnp.where(ev == lv2, ei, _BIG_I32))
            return lv1, li1, lv2, li2

        def exchange(par, lv1, li1, lv2, li2):
            locb[:] = jnp.where(
                lane0, lv1,
                jnp.where(lanes == 1, plsc.bitcast(li1, jnp.float32),
                          jnp.where(lanes == 2, lv2,
                                    plsc.bitcast(li2, jnp.float32))))
            pltpu.sync_copy(locb.at[pl.ds(0, 8)],
                            shared.at[par & 1, pl.ds(w * 8, 8)])
            plsc.subcore_barrier()
            pltpu.sync_copy(shared.at[par & 1], gbuf)
            v1 = plsc.load_gather(gbuf, [stride8])
            x1 = plsc.bitcast(plsc.load_gather(gbuf, [stride8 + 1]), jnp.int32)
            v2 = plsc.load_gather(gbuf, [stride8 + 2])
            x2 = plsc.bitcast(plsc.load_gather(gbuf, [stride8 + 3]), jnp.int32)
            g1v = bcast_max(v1)
            g1i = bcast_min_i32(jnp.where(v1 == g1v, x1, _BIG_I32))
            isw = x1 == g1i
            ev = jnp.where(isw, v2, v1)
            ei = jnp.where(isw, x2, x1)
            g2v = bcast_max(ev)
            g2i = bcast_min_i32(jnp.where(ev == g2v, ei, _BIG_I32))
            ax1 = plsc.load_gather(vx1, [g1i])
            ay1 = plsc.load_gather(vy1, [g1i])
            ax2 = plsc.load_gather(vx2, [g1i])
            ay2 = plsc.load_gather(vy2, [g1i])
            qx1 = plsc.load_gather(vx1, [g2i])
            qy1 = plsc.load_gather(vy1, [g2i])
            qx2 = plsc.load_gather(vx2, [g2i])
            qy2 = plsc.load_gather(vy2, [g2i])
            ox = jnp.where(ax2 < qx2, ax2, qx2) - jnp.where(ax1 > qx1, ax1, qx1)
            oy = jnp.where(ay2 < qy2, ay2, qy2) - jnp.where(ay1 > qy1, ay1, qy1)
            disjoint = jnp.any((ox <= 0.0) | (oy <= 0.0))
            return g1i, g1v, g2i, g2v, disjoint, par + 1

        t2 = local_scan(None)
        bo, v, pbo, pv, pvalid, par = exchange(0, *t2)

        def cond_fn(state):
            t, _, v, *_ = state
            return (t < _N) & jnp.any(v >= _THR)

        def body(state):
            t, bo, v, pbo, pv, pvalid, par = state
            plsc.store_scatter(vfin, [bo], v, mask=lane0)
            plsc.store_scatter(vm, [bo], jnp.full((_L,), -1.0, jnp.float32),
                               mask=lane0)
            bx1 = plsc.load_gather(vx1, [bo])
            by1 = plsc.load_gather(vy1, [bo])
            bx2 = plsc.load_gather(vx2, [bo])
            by2 = plsc.load_gather(vy2, [bo])
            a_i = (bx2 - bx1) * (by2 - by1)
            dargs = (bx1, by1, bx2, by2, a_i)

            def take_pending():
                decay_only(dargs)
                return pbo, pv, pbo, pv, jnp.zeros((), jnp.bool_), par

            def do_exchange():
                return exchange(par, *local_scan(dargs))

            return (t + 1,) + lax.cond(pvalid, take_pending, do_exchange)

        lax.while_loop(cond_fn, body,
                       (jnp.zeros((), jnp.int32), bo, v, pbo, pv, pvalid, par))

        pltpu.sync_copy(vfin.at[pl.ds(base, _E)], out.at[pl.ds(base, _E)])


_snms = functools.partial(
    pl.kernel,
    out_type=jax.ShapeDtypeStruct((_P,), jnp.float32),
    mesh=plsc.VectorSubcoreMesh(core_axis_name="c", subcore_axis_name="s",
                                num_cores=2, num_subcores=16),
    scratch_types=(
        [pltpu.VMEM((_P,), jnp.float32) for _ in range(7)]
        + [pltpu.VMEM((_L,), jnp.float32),
           pltpu.VMEM((_NT * 8,), jnp.float32),
           pltpu.VMEM_SHARED((2, _NT * 8), jnp.float32)]
    ),
    compiler_params=pltpu.CompilerParams(needs_layout_passes=False),
)(_snms_body)


@jax.jit
def kernel(boxes, scores):
    pad = _P - _N
    return _snms(
        jnp.pad(boxes[:, 0], (0, pad)),
        jnp.pad(boxes[:, 1], (0, pad)),
        jnp.pad(boxes[:, 2], (0, pad)),
        jnp.pad(boxes[:, 3], (0, pad)),
        jnp.pad(scores, (0, pad), constant_values=-1.0),
    )[:_N]


# speedup vs baseline: 1.1475x; 1.0069x over previous
import functools

import jax
import jax.numpy as jnp
from jax import lax
from jax.experimental import pallas as pl
from jax.experimental.pallas import tpu as pltpu
from jax.experimental.pallas import tpu_sc as plsc

_N = 1000
_P = 1024
_L = 16
_NT = 16
_E = _P // _NT
_CPT = _E // _L
_SIGMA = 0.5
_THR = 0.05
_BIG_I32 = 2**31 - 1


def _snms_body(hx1, hy1, hx2, hy2, hm, out,
               vx1, vy1, vx2, vy2, var, vm, vfin, locb, gbuf, shared):
    @pl.when(lax.axis_index("c") == 0)
    def _():
        w = lax.axis_index("s")
        base = w * _E

        pltpu.sync_copy(hx1, vx1)
        pltpu.sync_copy(hy1, vy1)
        pltpu.sync_copy(hx2, vx2)
        pltpu.sync_copy(hy2, vy2)
        pltpu.sync_copy(hm, vm)

        lanes = lax.iota(jnp.int32, _L)
        lane0 = lanes == 0
        stride8 = lanes * 8
        dnums = lax.GatherDimensionNumbers(
            offset_dims=(), collapsed_slice_dims=(0,), start_index_map=(0,))

        def perm(x, idx):
            return lax.gather(x, idx[:, None], dnums, (1,),
                              mode=lax.GatherScatterMode.PROMISE_IN_BOUNDS)

        lane15 = jnp.full((_L,), 15, jnp.int32)

        def bcast_max(x):
            return perm(plsc.cummax(x), lane15)

        def bcast_min_i32(x):
            return -perm(plsc.cummax(-x), lane15)

        zeros = jnp.zeros((_L,), jnp.float32)
        for k in range(_CPT):
            sl = pl.ds(base + k * _L, _L)
            var[sl] = (vx2[sl] - vx1[sl]) * (vy2[sl] - vy1[sl])
            vfin[sl] = zeros

        def decay_chunk(sl, decayed):
            bx1, by1, bx2, by2, a_i = decayed
            mc = vm[sl]
            cx1 = vx1[sl]
            cy1 = vy1[sl]
            cx2 = vx2[sl]
            cy2 = vy2[sl]
            xx1 = jnp.where(bx1 > cx1, bx1, cx1)
            yy1 = jnp.where(by1 > cy1, by1, cy1)
            xx2 = jnp.where(bx2 < cx2, bx2, cx2)
            yy2 = jnp.where(by2 < cy2, by2, cy2)
            dx = xx2 - xx1
            dy = yy2 - yy1
            inter = (jnp.where(dx > 0.0, dx, 0.0)
                     * jnp.where(dy > 0.0, dy, 0.0))
            iou = inter / (a_i + var[sl] - inter + 1e-7)
            dec = jnp.exp(iou * iou * (-1.0 / _SIGMA))
            mn = jnp.where(mc >= 0.0, mc * dec, mc)
            vm[sl] = mn
            return mn

        def decay_only(decayed):
            for k in range(_CPT):
                decay_chunk(pl.ds(base + k * _L, _L), decayed)

        def local_scan(decayed):
            b1 = jnp.full((_L,), -2.0, jnp.float32)
            i1 = jnp.zeros((_L,), jnp.int32)
            b2 = jnp.full((_L,), -3.0, jnp.float32)
            i2 = jnp.zeros((_L,), jnp.int32)
            for k in range(_CPT):
                off = base + k * _L
                sl = pl.ds(off, _L)
                if decayed is None:
                    mn = vm[sl]
                else:
                    mn = decay_chunk(sl, decayed)
                ix = lanes + off
                gt1 = mn > b1
                gt2 = mn > b2
                b2 = jnp.where(gt1, b1, jnp.where(gt2, mn, b2))
                i2 = jnp.where(gt1, i1, jnp.where(gt2, ix, i2))
                b1 = jnp.where(gt1, mn, b1)
                i1 = jnp.where(gt1, ix, i1)
            lv1 = bcast_max(b1)
            li1 = bcast_min_i32(jnp.where(b1 == lv1, i1, _BIG_I32))
            isw = i1 == li1
            ev = jnp.where(isw, b2, b1)
            ei = jnp.where(isw, i2, i1)
            lv2 = bcast_max(ev)
            li2 = bcast_min_i32(jnp.where(ev == lv2, ei, _BIG_I32))
            return lv1, li1, lv2, li2

        def exchange(par, lv1, li1, lv2, li2):
            locb[:] = jnp.where(
                lane0, lv1,
                jnp.where(lanes == 1, plsc.bitcast(li1, jnp.float32),
                          jnp.where(lanes == 2, lv2,
                                    plsc.bitcast(li2, jnp.float32))))
            pltpu.sync_copy(locb.at[pl.ds(0, 8)],
                            shared.at[par & 1, pl.ds(w * 8, 8)])
            plsc.subcore_barrier()
            pltpu.sync_copy(shared.at[par & 1], gbuf)
            v1 = plsc.load_gather(gbuf, [stride8])
            x1 = plsc.bitcast(plsc.load_gather(gbuf, [stride8 + 1]), jnp.int32)
            v2 = plsc.load_gather(gbuf, [stride8 + 2])
            x2 = plsc.bitcast(plsc.load_gather(gbuf, [stride8 + 3]), jnp.int32)
            g1v = bcast_max(v1)
            g1i = bcast_min_i32(jnp.where(v1 == g1v, x1, _BIG_I32))
            isw = x1 == g1i
            ev = jnp.where(isw, v2, v1)
            ei = jnp.where(isw, x2, x1)
            g2v = bcast_max(ev)
            g2i = bcast_min_i32(jnp.where(ev == g2v, ei, _BIG_I32))
            ax1 = plsc.load_gather(vx1, [g1i])
            ay1 = plsc.load_gather(vy1, [g1i])
            ax2 = plsc.load_gather(vx2, [g1i])
            ay2 = plsc.load_gather(vy2, [g1i])
            qx1 = plsc.load_gather(vx1, [g2i])
            qy1 = plsc.load_gather(vy1, [g2i])
            qx2 = plsc.load_gather(vx2, [g2i])
            qy2 = plsc.load_gather(vy2, [g2i])
            ox = jnp.where(ax2 < qx2, ax2, qx2) - jnp.where(ax1 > qx1, ax1, qx1)
            oy = jnp.where(ay2 < qy2, ay2, qy2) - jnp.where(ay1 > qy1, ay1, qy1)
            disjoint = jnp.any((ox <= 0.0) | (oy <= 0.0))
            return (g1i, g1v, (ax1, ay1, ax2, ay2),
                    g2i, g2v, (qx1, qy1, qx2, qy2), disjoint, par + 1)

        t2 = local_scan(None)
        bo, v, bcrd, pbo, pv, pcrd, pvalid, par = exchange(0, *t2)

        def cond_fn(state):
            t, _, v, *_ = state
            return (t < _N) & jnp.any(v >= _THR)

        def body(state):
            t, bo, v, bcrd, pbo, pv, pcrd, pvalid, par = state
            plsc.store_scatter(vfin, [bo], v, mask=lane0)
            plsc.store_scatter(vm, [bo], jnp.full((_L,), -1.0, jnp.float32),
                               mask=lane0)
            bx1, by1, bx2, by2 = bcrd
            a_i = (bx2 - bx1) * (by2 - by1)
            dargs = (bx1, by1, bx2, by2, a_i)

            def take_pending():
                decay_only(dargs)
                return (pbo, pv, pcrd, pbo, pv, pcrd,
                        jnp.zeros((), jnp.bool_), par)

            def do_exchange():
                return exchange(par, *local_scan(dargs))

            return (t + 1,) + lax.cond(pvalid, take_pending, do_exchange)

        lax.while_loop(cond_fn, body,
                       (jnp.zeros((), jnp.int32), bo, v, bcrd,
                        pbo, pv, pcrd, pvalid, par))

        pltpu.sync_copy(vfin.at[pl.ds(base, _E)], out.at[pl.ds(base, _E)])


_snms = functools.partial(
    pl.kernel,
    out_type=jax.ShapeDtypeStruct((_P,), jnp.float32),
    mesh=plsc.VectorSubcoreMesh(core_axis_name="c", subcore_axis_name="s",
                                num_cores=2, num_subcores=16),
    scratch_types=(
        [pltpu.VMEM((_P,), jnp.float32) for _ in range(7)]
        + [pltpu.VMEM((_L,), jnp.float32),
           pltpu.VMEM((_NT * 8,), jnp.float32),
           pltpu.VMEM_SHARED((2, _NT * 8), jnp.float32)]
    ),
    compiler_params=pltpu.CompilerParams(needs_layout_passes=False),
)(_snms_body)


@jax.jit
def kernel(boxes, scores):
    pad = _P - _N
    return _snms(
        jnp.pad(boxes[:, 0], (0, pad)),
        jnp.pad(boxes[:, 1], (0, pad)),
        jnp.pad(boxes[:, 2], (0, pad)),
        jnp.pad(boxes[:, 3], (0, pad)),
        jnp.pad(scores, (0, pad), constant_values=-1.0),
    )[:_N]
